# Initial kernel scaffold; baseline (speedup 1.0000x reference)
import jax
import jax.numpy as jnp
from jax.experimental import pallas as pl


def kernel(sentence, emb_table):
    raise NotImplementedError("write your pallas kernel here")


# speedup vs baseline: 3.5819x; 3.5819x over previous
import functools

import jax
import jax.numpy as jnp
from jax import lax
from jax.experimental import pallas as pl
from jax.experimental.pallas import tpu as pltpu
from jax.experimental.pallas import tpu_sc as plsc

_INFO = plsc.get_sparse_core_info()
_NC = _INFO.num_cores
_NS = _INFO.num_subcores
_NW = _NC * _NS

_CHUNK = 128
_NBUF = 4


def _emb_body(nchunk, sent_hbm, table_hbm, out_hbm,
              idx_v, rows_v, gsem, wsem):
  wid = lax.axis_index("s") * _NC + lax.axis_index("c")
  per_w = nchunk * _CHUNK
  base = wid * per_w

  pltpu.sync_copy(sent_hbm.at[wid], idx_v)

  def start_gather(c, b):
    pltpu.async_copy(table_hbm.at[idx_v.at[c]], rows_v.at[b], gsem.at[b])

  def wait_gather(b):
    pltpu.make_async_copy(table_hbm.at[idx_v.at[0]], rows_v.at[b],
                          gsem.at[b]).wait()

  def start_write(c, b):
    pltpu.async_copy(rows_v.at[b],
                     out_hbm.at[pl.ds(base + c * _CHUNK, _CHUNK)],
                     wsem.at[b])

  def wait_write(b):
    pltpu.make_async_copy(rows_v.at[b],
                          out_hbm.at[pl.ds(base, _CHUNK)],
                          wsem.at[b]).wait()

  for b in range(_NBUF):
    start_gather(b, b)

  def group(g, carry):
    for b in range(_NBUF):
      c = g * _NBUF + b
      wait_gather(b)
      start_write(c, b)
    for b in range(_NBUF):
      c = g * _NBUF + b
      wait_write(b)
      n = c + _NBUF

      @pl.when(n < nchunk)
      def _():
        start_gather(n, b)

    return carry

  lax.fori_loop(0, nchunk // _NBUF, group, None)


@jax.jit
def _run(sentence, emb_table):
  bsz, hist = sentence.shape
  vocab, dim = emb_table.shape
  n = bsz * hist
  assert n % (_NW * _CHUNK) == 0
  per_w = n // _NW
  nchunk = per_w // _CHUNK
  assert nchunk % _NBUF == 0

  sent = sentence.reshape(_NW, nchunk, _CHUNK)

  out = pl.kernel(
      functools.partial(_emb_body, nchunk),
      out_type=jax.ShapeDtypeStruct((n, dim), jnp.float32),
      mesh=plsc.VectorSubcoreMesh(core_axis_name="c", subcore_axis_name="s"),
      compiler_params=pltpu.CompilerParams(use_tc_tiling_on_sc=False),
      scratch_types=[
          pltpu.VMEM((nchunk, _CHUNK), jnp.int32),
          pltpu.VMEM((_NBUF, _CHUNK, dim), jnp.float32),
          pltpu.SemaphoreType.DMA((_NBUF,)),
          pltpu.SemaphoreType.DMA((_NBUF,)),
      ],
  )(sent, emb_table)
  return out.reshape(bsz, hist, dim)


def kernel(sentence, emb_table):
  return _run(sentence, emb_table)


# speedup vs baseline: 3.5827x; 1.0002x over previous
import functools

import jax
import jax.numpy as jnp
from jax import lax
from jax.experimental import pallas as pl
from jax.experimental.pallas import tpu as pltpu
from jax.experimental.pallas import tpu_sc as plsc

_INFO = plsc.get_sparse_core_info()
_NC = _INFO.num_cores
_NS = _INFO.num_subcores
_NW = _NC * _NS

_NBUF = 4

_SPLITS = ((0, 128), (128, 72))


def _emb_body(per_w, hist, sent_hbm, table_hbm, out_hbm,
              idx_v, rows_v, gsem, wsem):
  wid = lax.axis_index("s") * _NC + lax.axis_index("c")
  base = wid * per_w

  pltpu.sync_copy(sent_hbm.at[pl.ds(base, per_w)], idx_v)

  def start_gather(s, b):
    for off, cnt in _SPLITS:
      pltpu.async_copy(table_hbm.at[idx_v.at[s, pl.ds(off, cnt)]],
                       rows_v.at[b, pl.ds(off, cnt)], gsem.at[b])

  def wait_gather(b):
    pltpu.make_async_copy(table_hbm.at[idx_v.at[0, pl.ds(0, hist)]],
                          rows_v.at[b], gsem.at[b]).wait()

  def start_write(s, b):
    pltpu.async_copy(rows_v.at[b], out_hbm.at[base + s], wsem.at[b])

  def wait_write(b):
    pltpu.make_async_copy(rows_v.at[b], out_hbm.at[base], wsem.at[b]).wait()

  for b in range(_NBUF):
    start_gather(b, b)

  def group(g, carry):
    for b in range(_NBUF):
      s = g * _NBUF + b
      wait_gather(b)
      start_write(s, b)
    for b in range(_NBUF):
      s = g * _NBUF + b
      wait_write(b)
      n = s + _NBUF

      @pl.when(n < per_w)
      def _():
        start_gather(n, b)

    return carry

  lax.fori_loop(0, per_w // _NBUF, group, None)


@jax.jit
def _run(sentence, emb_table):
  bsz, hist = sentence.shape
  vocab, dim = emb_table.shape
  assert bsz % _NW == 0
  per_w = bsz // _NW
  assert per_w % _NBUF == 0
  assert hist == sum(c for _, c in _SPLITS)

  out = pl.kernel(
      functools.partial(_emb_body, per_w, hist),
      out_type=jax.ShapeDtypeStruct((bsz, hist, dim), jnp.float32),
      mesh=plsc.VectorSubcoreMesh(core_axis_name="c", subcore_axis_name="s"),
      compiler_params=pltpu.CompilerParams(use_tc_tiling_on_sc=False),
      scratch_types=[
          pltpu.VMEM((per_w, hist), jnp.int32),
          pltpu.VMEM((_NBUF, hist, dim), jnp.float32),
          pltpu.SemaphoreType.DMA((_NBUF,)),
          pltpu.SemaphoreType.DMA((_NBUF,)),
      ],
  )(sentence, emb_table)
  return out


def kernel(sentence, emb_table):
  return _run(sentence, emb_table)


# speedup vs baseline: 5.5735x; 1.5557x over previous
import functools

import jax
import jax.numpy as jnp
from jax import lax
from jax.experimental import pallas as pl
from jax.experimental.pallas import tpu as pltpu
from jax.experimental.pallas import tpu_sc as plsc

_INFO = plsc.get_sparse_core_info()
_NC = _INFO.num_cores
_NS = _INFO.num_subcores
_NW = _NC * _NS

_CHUNK = 128
_NBUF = 4
_LANES = 128


def _emb_body(nchunk, dim, sent_hbm, table_hbm, out_hbm,
              idx_v, rows_v, gsem, wsem):
  wid = lax.axis_index("s") * _NC + lax.axis_index("c")
  per_w = nchunk * _CHUNK
  base = wid * per_w

  pltpu.sync_copy(sent_hbm.at[pl.ds(wid * nchunk, nchunk)], idx_v)

  def start_gather(c, b):
    pltpu.async_copy(table_hbm.at[idx_v.at[c]], rows_v.at[b], gsem.at[b])

  def wait_gather(b):
    pltpu.make_async_copy(table_hbm.at[idx_v.at[0]], rows_v.at[b],
                          gsem.at[b]).wait()

  def start_write(c, b):
    pltpu.async_copy(rows_v.at[b],
                     out_hbm.at[pl.ds(base + c * _CHUNK, _CHUNK),
                                pl.ds(0, dim)],
                     wsem.at[b])

  def wait_write(b):
    pltpu.make_async_copy(rows_v.at[b],
                          out_hbm.at[pl.ds(base, _CHUNK), pl.ds(0, dim)],
                          wsem.at[b]).wait()

  for b in range(_NBUF):
    start_gather(b, b)

  def group(g, carry):
    for b in range(_NBUF):
      c = g * _NBUF + b
      wait_gather(b)
      start_write(c, b)
    for b in range(_NBUF):
      c = g * _NBUF + b
      wait_write(b)
      n = c + _NBUF

      @pl.when(n < nchunk)
      def _():
        start_gather(n, b)

    return carry

  lax.fori_loop(0, nchunk // _NBUF, group, None)


@jax.jit
def _run(sentence, emb_table):
  bsz, hist = sentence.shape
  vocab, dim = emb_table.shape
  n = bsz * hist
  assert n % (_NW * _CHUNK) == 0
  per_w = n // _NW
  nchunk_w = per_w // _CHUNK
  assert nchunk_w % _NBUF == 0

  sent = sentence.reshape(_NW * nchunk_w, _CHUNK)

  out = pl.kernel(
      functools.partial(_emb_body, nchunk_w, dim),
      out_type=jax.ShapeDtypeStruct((n, _LANES), jnp.float32),
      mesh=plsc.VectorSubcoreMesh(core_axis_name="c", subcore_axis_name="s"),
      compiler_params=pltpu.CompilerParams(use_tc_tiling_on_sc=False),
      scratch_types=[
          pltpu.VMEM((nchunk_w, _CHUNK), jnp.int32),
          pltpu.VMEM((_NBUF, _CHUNK, dim), jnp.float32),
          pltpu.SemaphoreType.DMA((_NBUF,)),
          pltpu.SemaphoreType.DMA((_NBUF,)),
      ],
  )(sent, emb_table)
  return out[:, :dim].reshape(bsz, hist, dim)


def kernel(sentence, emb_table):
  return _run(sentence, emb_table)


# speedup vs baseline: 10.3887x; 1.8639x over previous
import functools

import jax
import jax.numpy as jnp
from jax import lax
from jax.experimental import pallas as pl
from jax.experimental.pallas import tpu as pltpu
from jax.experimental.pallas import tpu_sc as plsc

_INFO = plsc.get_sparse_core_info()
_NC = _INFO.num_cores
_NS = _INFO.num_subcores
_NW = _NC * _NS

_CHUNK = 128
_NBUF = 4
_LANES = 128


def _emb_body(nchunk, dim, sent_hbm, table_hbm, out_hbm,
              idx_v, table_sh, rows_v, gsem, wsem):
  sid = lax.axis_index("s")
  wid = sid * _NC + lax.axis_index("c")
  per_w = nchunk * _CHUNK
  base = wid * per_w

  @pl.when(sid == 0)
  def _():
    pltpu.sync_copy(table_hbm, table_sh)

  pltpu.sync_copy(sent_hbm.at[pl.ds(wid * nchunk, nchunk)], idx_v)
  plsc.subcore_barrier()

  def start_gather(c, b):
    pltpu.async_copy(table_sh.at[idx_v.at[c]], rows_v.at[b], gsem.at[b])

  def wait_gather(b):
    pltpu.make_async_copy(table_sh.at[idx_v.at[0]], rows_v.at[b],
                          gsem.at[b]).wait()

  def start_write(c, b):
    pltpu.async_copy(rows_v.at[b],
                     out_hbm.at[pl.ds(base + c * _CHUNK, _CHUNK),
                                pl.ds(0, dim)],
                     wsem.at[b])

  def wait_write(b):
    pltpu.make_async_copy(rows_v.at[b],
                          out_hbm.at[pl.ds(base, _CHUNK), pl.ds(0, dim)],
                          wsem.at[b]).wait()

  for b in range(_NBUF):
    start_gather(b, b)

  def group(g, carry):
    for b in range(_NBUF):
      c = g * _NBUF + b
      wait_gather(b)
      start_write(c, b)
    for b in range(_NBUF):
      c = g * _NBUF + b
      wait_write(b)
      n = c + _NBUF

      @pl.when(n < nchunk)
      def _():
        start_gather(n, b)

    return carry

  lax.fori_loop(0, nchunk // _NBUF, group, None)


@jax.jit
def _run(sentence, emb_table):
  bsz, hist = sentence.shape
  vocab, dim = emb_table.shape
  n = bsz * hist
  assert n % (_NW * _CHUNK) == 0
  per_w = n // _NW
  nchunk_w = per_w // _CHUNK
  assert nchunk_w % _NBUF == 0

  sent = sentence.reshape(_NW * nchunk_w, _CHUNK)

  out = pl.kernel(
      functools.partial(_emb_body, nchunk_w, dim),
      out_type=jax.ShapeDtypeStruct((n, _LANES), jnp.float32),
      mesh=plsc.VectorSubcoreMesh(core_axis_name="c", subcore_axis_name="s"),
      compiler_params=pltpu.CompilerParams(use_tc_tiling_on_sc=False),
      scratch_types=[
          pltpu.VMEM((nchunk_w, _CHUNK), jnp.int32),
          pltpu.VMEM_SHARED((vocab, dim), jnp.float32),
          pltpu.VMEM((_NBUF, _CHUNK, dim), jnp.float32),
          pltpu.SemaphoreType.DMA((_NBUF,)),
          pltpu.SemaphoreType.DMA((_NBUF,)),
      ],
  )(sent, emb_table)
  return out[:, :dim].reshape(bsz, hist, dim)


def kernel(sentence, emb_table):
  return _run(sentence, emb_table)


# speedup vs baseline: 10.4343x; 1.0044x over previous
import functools

import jax
import jax.numpy as jnp
from jax import lax
from jax.experimental import pallas as pl
from jax.experimental.pallas import tpu as pltpu
from jax.experimental.pallas import tpu_sc as plsc

_INFO = plsc.get_sparse_core_info()
_NC = _INFO.num_cores
_NS = _INFO.num_subcores
_NW = _NC * _NS

_CHUNK = 128
_NBUF = 8
_LANES = 128


def _emb_body(nchunk, dim, sent_hbm, table_hbm, out_hbm,
              idx_v, table_sh, rows_v, gsem, wsem):
  sid = lax.axis_index("s")
  wid = sid * _NC + lax.axis_index("c")
  per_w = nchunk * _CHUNK
  base = wid * per_w

  @pl.when(sid == 0)
  def _():
    pltpu.sync_copy(table_hbm, table_sh)

  pltpu.sync_copy(sent_hbm.at[pl.ds(wid * nchunk, nchunk)], idx_v)
  plsc.subcore_barrier()

  def start_gather(c, b):
    pltpu.async_copy(table_sh.at[idx_v.at[c]], rows_v.at[b], gsem.at[b])

  def wait_gather(b):
    pltpu.make_async_copy(table_sh.at[idx_v.at[0]], rows_v.at[b],
                          gsem.at[b]).wait()

  def start_write(c, b):
    pltpu.async_copy(rows_v.at[b],
                     out_hbm.at[pl.ds(base + c * _CHUNK, _CHUNK),
                                pl.ds(0, dim)],
                     wsem.at[b])

  def wait_write(b):
    pltpu.make_async_copy(rows_v.at[b],
                          out_hbm.at[pl.ds(base, _CHUNK), pl.ds(0, dim)],
                          wsem.at[b]).wait()

  for b in range(_NBUF):
    start_gather(b, b)

  def group(g, carry):
    for b in range(_NBUF):
      c = g * _NBUF + b
      wait_gather(b)
      start_write(c, b)
    for b in range(_NBUF):
      c = g * _NBUF + b
      wait_write(b)
      n = c + _NBUF

      @pl.when(n < nchunk)
      def _():
        start_gather(n, b)

    return carry

  lax.fori_loop(0, nchunk // _NBUF, group, None)


@jax.jit
def _run(sentence, emb_table):
  bsz, hist = sentence.shape
  vocab, dim = emb_table.shape
  n = bsz * hist
  assert n % (_NW * _CHUNK) == 0
  per_w = n // _NW
  nchunk_w = per_w // _CHUNK
  assert nchunk_w % _NBUF == 0

  sent = sentence.reshape(_NW * nchunk_w, _CHUNK)

  out = pl.kernel(
      functools.partial(_emb_body, nchunk_w, dim),
      out_type=jax.ShapeDtypeStruct((n, _LANES), jnp.float32),
      mesh=plsc.VectorSubcoreMesh(core_axis_name="c", subcore_axis_name="s"),
      compiler_params=pltpu.CompilerParams(use_tc_tiling_on_sc=False),
      scratch_types=[
          pltpu.VMEM((nchunk_w, _CHUNK), jnp.int32),
          pltpu.VMEM_SHARED((vocab, dim), jnp.float32),
          pltpu.VMEM((_NBUF, _CHUNK, dim), jnp.float32),
          pltpu.SemaphoreType.DMA((_NBUF,)),
          pltpu.SemaphoreType.DMA((_NBUF,)),
      ],
  )(sent, emb_table)
  return out[:, :dim].reshape(bsz, hist, dim)


def kernel(sentence, emb_table):
  return _run(sentence, emb_table)
